# Initial kernel scaffold; baseline (speedup 1.0000x reference)
import jax
import jax.numpy as jnp
from jax.experimental import pallas as pl


def kernel(x, edge_index, edge_vals, W, b):
    raise NotImplementedError("write your pallas kernel here")


# speedup vs baseline: 4.4815x; 4.4815x over previous
import functools

import jax
import jax.numpy as jnp
from jax import lax
from jax.experimental import pallas as pl
from jax.experimental.pallas import tpu as pltpu
from jax.experimental.pallas import tpu_sc as plsc

N_NODES = 10000
N_EDGES = 320000
D = 128
L = 16

NC = 2
NS = 16
NW = NC * NS
EPW = N_EDGES // NW
K = 80
CHUNKS = EPW // K
RPT = 624
TAIL = N_NODES - RPT * NS
ZROWS = 208


def _sc_scatter(x, src, dst, vals):
    mesh = plsc.VectorSubcoreMesh(core_axis_name="c", subcore_axis_name="s")

    @functools.partial(
        pl.kernel,
        out_type=jax.ShapeDtypeStruct((NC, N_NODES, D), jnp.float32),
        mesh=mesh,
        scratch_types=[
            pltpu.VMEM((K,), jnp.int32),
            pltpu.VMEM((K,), jnp.int32),
            pltpu.VMEM((K,), jnp.float32),
            pltpu.VMEM((K, D), jnp.float32),
            pltpu.VMEM((ZROWS, D), jnp.float32),
            pltpu.VMEM_SHARED((N_NODES, D), jnp.float32),
            pltpu.SemaphoreType.DMA,
        ],
    )
    def k(x_hbm, src_hbm, dst_hbm, vals_hbm, out_hbm,
          src_v, dst_v, vals_v, rows_v, zero_v, acc_sh, sem):
        cid = lax.axis_index("c")
        sid = lax.axis_index("s")
        wid = sid * NC + cid

        zvec = jnp.zeros((L,), jnp.float32)

        def zbody(i, _):
            for j in range(D // L):
                zero_v[i, pl.ds(j * L, L)] = zvec
            return 0

        lax.fori_loop(0, ZROWS, zbody, 0)
        row0 = pl.multiple_of(sid * RPT, 8)
        for t in range(RPT // ZROWS):
            pltpu.sync_copy(zero_v, acc_sh.at[pl.ds(row0 + t * ZROWS, ZROWS)])

        @pl.when(sid == NS - 1)
        def _zero_tail():
            pltpu.sync_copy(zero_v.at[pl.ds(0, TAIL)],
                            acc_sh.at[pl.ds(RPT * NS, TAIL)])

        plsc.subcore_barrier()

        base0 = wid * EPW

        def chunk(c, _):
            base = base0 + c * K
            pltpu.sync_copy(src_hbm.at[pl.ds(base, K)], src_v)
            pltpu.sync_copy(dst_hbm.at[pl.ds(base, K)], dst_v)
            pltpu.sync_copy(vals_hbm.at[pl.ds(base, K)], vals_v)
            pltpu.async_copy(x_hbm.at[src_v], rows_v, sem).wait()

            def scale(g, _):
                vv = vals_v[pl.ds(g * L, L)]
                for i in range(L):
                    v = vv[i]
                    e = g * L + i
                    for j in range(D // L):
                        sl = pl.ds(j * L, L)
                        rows_v[e, sl] = rows_v[e, sl] * v
                return 0

            lax.fori_loop(0, K // L, scale, 0)
            pltpu.sync_copy(rows_v, acc_sh.at[dst_v], add=True)
            return 0

        lax.fori_loop(0, CHUNKS, chunk, 0)
        plsc.subcore_barrier()

        row0w = pl.multiple_of(sid * RPT, 8)
        pltpu.sync_copy(acc_sh.at[pl.ds(row0w, RPT)],
                        out_hbm.at[cid, pl.ds(row0w, RPT)])

        @pl.when(sid == NS - 1)
        def _write_tail():
            pltpu.sync_copy(acc_sh.at[pl.ds(RPT * NS, TAIL)],
                            out_hbm.at[cid, pl.ds(RPT * NS, TAIL)])

    return k(x, src, dst, vals)


def _tc_combine(partials, W, b):
    R = 1000

    def body(p0_ref, p1_ref, w_ref, b_ref, o_ref):
        s = p0_ref[...] + p1_ref[...]
        y = jnp.dot(s, w_ref[...], preferred_element_type=jnp.float32)
        o_ref[...] = jnp.maximum(y + b_ref[...], 0.0)

    return pl.pallas_call(
        body,
        grid=(N_NODES // R,),
        in_specs=[
            pl.BlockSpec((R, D), lambda i: (i, 0)),
            pl.BlockSpec((R, D), lambda i: (i, 0)),
            pl.BlockSpec((D, D), lambda i: (0, 0)),
            pl.BlockSpec((1, D), lambda i: (0, 0)),
        ],
        out_specs=pl.BlockSpec((R, D), lambda i: (i, 0)),
        out_shape=jax.ShapeDtypeStruct((N_NODES, D), jnp.float32),
    )(partials[0], partials[1], W, b.reshape(1, D))


def kernel(x, edge_index, edge_vals, W, b):
    src = edge_index[0].astype(jnp.int32)
    dst = edge_index[1].astype(jnp.int32)
    partials = _sc_scatter(x, src, dst, edge_vals.astype(jnp.float32))
    return _tc_combine(partials, W, b)


# speedup vs baseline: 10.7792x; 2.4053x over previous
import functools

import jax
import jax.numpy as jnp
from jax import lax
from jax.experimental import pallas as pl
from jax.experimental.pallas import tpu as pltpu
from jax.experimental.pallas import tpu_sc as plsc

N_NODES = 10000
N_EDGES = 320000
D = 128
L = 16

NC = 2
NS = 16
NW = NC * NS
EPW = N_EDGES // NW
K = 80
CHUNKS = EPW // K
PAIRS = (CHUNKS - 1) // 2
RPT = 624
TAIL = N_NODES - RPT * NS
ZROWS = 16


def _sc_scatter(x, src, dst, vals):
    mesh = plsc.VectorSubcoreMesh(core_axis_name="c", subcore_axis_name="s")

    @functools.partial(
        pl.kernel,
        out_type=jax.ShapeDtypeStruct((NC, N_NODES, D), jnp.float32),
        mesh=mesh,
        scratch_types=[
            pltpu.VMEM((EPW,), jnp.int32),
            pltpu.VMEM((EPW,), jnp.float32),
            pltpu.VMEM((K,), jnp.int32),
            pltpu.VMEM((K,), jnp.int32),
            pltpu.VMEM((K, D), jnp.float32),
            pltpu.VMEM((K, D), jnp.float32),
            pltpu.VMEM((ZROWS, D), jnp.float32),
            pltpu.VMEM_SHARED((N_NODES, D), jnp.float32),
            pltpu.SemaphoreType.DMA,
            pltpu.SemaphoreType.DMA,
            pltpu.SemaphoreType.DMA,
            pltpu.SemaphoreType.DMA,
        ],
    )
    def k(x_hbm, src_hbm, dst_hbm, vals_hbm, out_hbm,
          src_v, vals_v, dst0_v, dst1_v, rows0_v, rows1_v, zero_v, acc_sh,
          rsem0, rsem1, dsem0, dsem1):
        cid = lax.axis_index("c")
        sid = lax.axis_index("s")
        wid = sid * NC + cid
        base0 = wid * EPW

        pltpu.sync_copy(src_hbm.at[pl.ds(base0, EPW)], src_v)
        pltpu.sync_copy(vals_hbm.at[pl.ds(base0, EPW)], vals_v)

        dbuf = (dst0_v, dst1_v)
        dsem = (dsem0, dsem1)

        def dst_fetch(c, par):
            pltpu.async_copy(dst_hbm.at[pl.ds(base0 + c * K, K)],
                             dbuf[par], dsem[par])

        def dst_wait(c, par):
            pltpu.make_async_copy(dst_hbm.at[pl.ds(base0 + c * K, K)],
                                  dbuf[par], dsem[par]).wait()

        dst_fetch(0, 0)
        dst_fetch(1, 1)

        zvec = jnp.zeros((L,), jnp.float32)
        for j in range(D // L):
            for i in range(ZROWS):
                zero_v[i, pl.ds(j * L, L)] = zvec
        row0 = pl.multiple_of(sid * RPT, 8)

        def zcopy(t, _):
            off = pl.multiple_of(row0 + t * ZROWS, 8)
            pltpu.sync_copy(zero_v, acc_sh.at[pl.ds(off, ZROWS)])
            return 0

        lax.fori_loop(0, RPT // ZROWS, zcopy, 0)

        @pl.when(sid == NS - 1)
        def _zero_tail():
            pltpu.sync_copy(zero_v, acc_sh.at[pl.ds(RPT * NS, TAIL)])

        plsc.subcore_barrier()

        rbuf = (rows0_v, rows1_v)
        rsem = (rsem0, rsem1)

        def gather(c, par):
            pltpu.async_copy(
                x_hbm.at[src_v.at[pl.ds(c * K, K)]], rbuf[par], rsem[par])

        def gather_wait(c, par):
            pltpu.make_async_copy(
                x_hbm.at[src_v.at[pl.ds(c * K, K)]], rbuf[par],
                rsem[par]).wait()

        def process(c, par):
            buf = rbuf[par]

            def scale(g, _):
                vv = vals_v[pl.ds(c * K + g * L, L)]
                for i in range(L):
                    v = vv[i]
                    e = g * L + i
                    for j in range(D // L):
                        sl = pl.ds(j * L, L)
                        buf[e, sl] = buf[e, sl] * v
                return 0

            lax.fori_loop(0, K // L, scale, 0)
            dst_wait(c, par)
            pltpu.sync_copy(buf, acc_sh.at[dbuf[par]], add=True)

        gather(0, 0)

        def pair(p, _):
            c0 = 2 * p
            gather(c0 + 1, 1)
            gather_wait(c0, 0)
            process(c0, 0)
            dst_fetch(c0 + 2, 0)
            gather(c0 + 2, 0)
            gather_wait(c0 + 1, 1)
            process(c0 + 1, 1)

            @pl.when(c0 + 3 < CHUNKS)
            def _prefetch_next_odd():
                dst_fetch(c0 + 3, 1)

            return 0

        lax.fori_loop(0, PAIRS, pair, 0)
        gather_wait(CHUNKS - 1, 0)
        process(CHUNKS - 1, 0)
        plsc.subcore_barrier()

        pltpu.sync_copy(acc_sh.at[pl.ds(row0, RPT)],
                        out_hbm.at[cid, pl.ds(row0, RPT)])

        @pl.when(sid == NS - 1)
        def _write_tail():
            pltpu.sync_copy(acc_sh.at[pl.ds(RPT * NS, TAIL)],
                            out_hbm.at[cid, pl.ds(RPT * NS, TAIL)])

    return k(x, src, dst, vals)


def _tc_combine(partials, W, b):
    R = 1000

    def body(p0_ref, p1_ref, w_ref, b_ref, o_ref):
        s = p0_ref[...] + p1_ref[...]
        y = jnp.dot(s, w_ref[...], preferred_element_type=jnp.float32)
        o_ref[...] = jnp.maximum(y + b_ref[...], 0.0)

    return pl.pallas_call(
        body,
        grid=(N_NODES // R,),
        in_specs=[
            pl.BlockSpec((R, D), lambda i: (i, 0)),
            pl.BlockSpec((R, D), lambda i: (i, 0)),
            pl.BlockSpec((D, D), lambda i: (0, 0)),
            pl.BlockSpec((1, D), lambda i: (0, 0)),
        ],
        out_specs=pl.BlockSpec((R, D), lambda i: (i, 0)),
        out_shape=jax.ShapeDtypeStruct((N_NODES, D), jnp.float32),
    )(partials[0], partials[1], W, b.reshape(1, D))


def kernel(x, edge_index, edge_vals, W, b):
    src = edge_index[0].astype(jnp.int32)
    dst = edge_index[1].astype(jnp.int32)
    partials = _sc_scatter(x, src, dst, edge_vals.astype(jnp.float32))
    return _tc_combine(partials, W, b)


# speedup vs baseline: 10.8657x; 1.0080x over previous
import functools

import jax
import jax.numpy as jnp
from jax import lax
from jax.experimental import pallas as pl
from jax.experimental.pallas import tpu as pltpu
from jax.experimental.pallas import tpu_sc as plsc

N_NODES = 10000
N_EDGES = 320000
D = 128
L = 16

NC = 2
NS = 16
NW = NC * NS
EPW = N_EDGES // NW
K = 80
CHUNKS = EPW // K
NBUF = 4
STEPS = (CHUNKS - 1) // NBUF
RPT = 624
TAIL = N_NODES - RPT * NS
ZROWS = 16


def _sc_scatter(x, src, dst, vals):
    mesh = plsc.VectorSubcoreMesh(core_axis_name="c", subcore_axis_name="s")

    rows_t = [pltpu.VMEM((K, D), jnp.float32) for _ in range(NBUF)]
    srcb_t = [pltpu.VMEM((K,), jnp.int32) for _ in range(NBUF)]
    dstb_t = [pltpu.VMEM((K,), jnp.int32) for _ in range(NBUF)]
    valb_t = [pltpu.VMEM((K,), jnp.float32) for _ in range(NBUF)]
    sems_t = [pltpu.SemaphoreType.DMA for _ in range(3 * NBUF)]

    @functools.partial(
        pl.kernel,
        out_type=jax.ShapeDtypeStruct((NC, N_NODES, D), jnp.float32),
        mesh=mesh,
        scratch_types=(rows_t + srcb_t + dstb_t + valb_t
                       + [pltpu.VMEM((ZROWS, D), jnp.float32),
                          pltpu.VMEM_SHARED((N_NODES, D), jnp.float32)]
                       + sems_t),
    )
    def k(x_hbm, src_hbm, dst_hbm, vals_hbm, out_hbm, *refs):
        rows = refs[0:NBUF]
        srcb = refs[NBUF:2 * NBUF]
        dstb = refs[2 * NBUF:3 * NBUF]
        valb = refs[3 * NBUF:4 * NBUF]
        zero_v = refs[4 * NBUF]
        acc_sh = refs[4 * NBUF + 1]
        rsem = refs[4 * NBUF + 2:4 * NBUF + 2 + NBUF]
        ssem = refs[4 * NBUF + 2 + NBUF:4 * NBUF + 2 + 2 * NBUF]
        msem = refs[4 * NBUF + 2 + 2 * NBUF:4 * NBUF + 2 + 3 * NBUF]

        cid = lax.axis_index("c")
        sid = lax.axis_index("s")
        wid = sid * NC + cid
        base0 = wid * EPW

        def meta_fetch(c, q):
            sl = pl.ds(base0 + c * K, K)
            pltpu.async_copy(src_hbm.at[sl], srcb[q], msem[q])
            pltpu.async_copy(dst_hbm.at[sl], dstb[q], msem[q])
            pltpu.async_copy(vals_hbm.at[sl], valb[q], msem[q])

        def meta_wait(c, q):
            sl = pl.ds(base0 + c * K, K)
            pltpu.make_async_copy(src_hbm.at[sl], srcb[q], msem[q]).wait()
            pltpu.make_async_copy(dst_hbm.at[sl], dstb[q], msem[q]).wait()
            pltpu.make_async_copy(vals_hbm.at[sl], valb[q], msem[q]).wait()

        def gather(q):
            pltpu.async_copy(x_hbm.at[srcb[q]], rows[q], rsem[q])

        def gather_wait(q):
            pltpu.make_async_copy(x_hbm.at[srcb[q]], rows[q],
                                  rsem[q]).wait()

        def scatter(q):
            pltpu.async_copy(rows[q], acc_sh.at[dstb[q]], ssem[q], add=True)

        def scatter_wait(q):
            pltpu.make_async_copy(rows[q], acc_sh.at[dstb[q]],
                                  ssem[q]).wait()

        def scale(q):
            buf = rows[q]
            vbuf = valb[q]

            def body(g, _):
                vv = vbuf[pl.ds(g * L, L)]
                for i in range(L):
                    v = vv[i]
                    e = g * L + i
                    for j in range(D // L):
                        sl = pl.ds(j * L, L)
                        buf[e, sl] = buf[e, sl] * v
                return 0

            lax.fori_loop(0, K // L, body, 0)

        meta_fetch(0, 0)
        meta_fetch(1, 1)

        zvec = jnp.zeros((L,), jnp.float32)
        for j in range(D // L):
            for i in range(ZROWS):
                zero_v[i, pl.ds(j * L, L)] = zvec
        row0 = pl.multiple_of(sid * RPT, 8)

        def zcopy(t, _):
            off = pl.multiple_of(row0 + t * ZROWS, 8)
            pltpu.sync_copy(zero_v, acc_sh.at[pl.ds(off, ZROWS)])
            return 0

        lax.fori_loop(0, RPT // ZROWS, zcopy, 0)

        @pl.when(sid == NS - 1)
        def _zero_tail():
            pltpu.sync_copy(zero_v, acc_sh.at[pl.ds(RPT * NS, TAIL)])

        meta_wait(0, 0)
        gather(0)
        meta_wait(1, 1)
        gather(1)
        plsc.subcore_barrier()

        def step(s, _):
            c0 = s * NBUF
            for q in range(NBUF):
                c = c0 + q
                f = (q + 2) % NBUF

                @pl.when(c + 2 < CHUNKS)
                def _prefetch():
                    meta_fetch(c + 2, f)

                gather_wait(q)
                scale(q)
                scatter(q)
                scatter_wait(q)

                @pl.when(c + 2 < CHUNKS)
                def _launch():
                    meta_wait(c + 2, f)
                    gather(f)

            return 0

        lax.fori_loop(0, STEPS, step, 0)

        last = CHUNKS - 1
        gather_wait(last % NBUF)
        scale(last % NBUF)
        scatter(last % NBUF)
        scatter_wait(last % NBUF)
        plsc.subcore_barrier()

        pltpu.sync_copy(acc_sh.at[pl.ds(row0, RPT)],
                        out_hbm.at[cid, pl.ds(row0, RPT)])

        @pl.when(sid == NS - 1)
        def _write_tail():
            pltpu.sync_copy(acc_sh.at[pl.ds(RPT * NS, TAIL)],
                            out_hbm.at[cid, pl.ds(RPT * NS, TAIL)])

    return k(x, src, dst, vals)


def _tc_combine(partials, W, b):
    R = 1000

    def body(p0_ref, p1_ref, w_ref, b_ref, o_ref):
        s = p0_ref[...] + p1_ref[...]
        y = jnp.dot(s, w_ref[...], preferred_element_type=jnp.float32)
        o_ref[...] = jnp.maximum(y + b_ref[...], 0.0)

    return pl.pallas_call(
        body,
        grid=(N_NODES // R,),
        in_specs=[
            pl.BlockSpec((R, D), lambda i: (i, 0)),
            pl.BlockSpec((R, D), lambda i: (i, 0)),
            pl.BlockSpec((D, D), lambda i: (0, 0)),
            pl.BlockSpec((1, D), lambda i: (0, 0)),
        ],
        out_specs=pl.BlockSpec((R, D), lambda i: (i, 0)),
        out_shape=jax.ShapeDtypeStruct((N_NODES, D), jnp.float32),
    )(partials[0], partials[1], W, b.reshape(1, D))


def kernel(x, edge_index, edge_vals, W, b):
    src = edge_index[0].astype(jnp.int32)
    dst = edge_index[1].astype(jnp.int32)
    partials = _sc_scatter(x, src, dst, edge_vals.astype(jnp.float32))
    return _tc_combine(partials, W, b)


# speedup vs baseline: 12.0306x; 1.1072x over previous
import functools

import jax
import jax.numpy as jnp
from jax import lax
from jax.experimental import pallas as pl
from jax.experimental.pallas import tpu as pltpu
from jax.experimental.pallas import tpu_sc as plsc

N_NODES = 10000
N_EDGES = 320000
D = 128
L = 16

NC = 2
NS = 16
NW = NC * NS
EPW = N_EDGES // NW
K = 80
CHUNKS = EPW // K
NBUF = 4
STEPS = (CHUNKS - 1) // NBUF
RPT = 624
TAIL = N_NODES - RPT * NS
ZROWS = 16


def _sc_scatter(x, src, dst, vals):
    mesh = plsc.VectorSubcoreMesh(core_axis_name="c", subcore_axis_name="s")

    rows_t = [pltpu.VMEM((K, D), jnp.float32) for _ in range(NBUF)]
    srcb_t = [pltpu.VMEM((K,), jnp.int32) for _ in range(NBUF)]
    dstb_t = [pltpu.VMEM((K,), jnp.int32) for _ in range(NBUF)]
    valb_t = [pltpu.VMEM((K,), jnp.float32) for _ in range(NBUF)]
    sems_t = [pltpu.SemaphoreType.DMA for _ in range(3 * NBUF)]

    @functools.partial(
        pl.kernel,
        out_type=jax.ShapeDtypeStruct((NC, N_NODES, D), jnp.float32),
        mesh=mesh,
        scratch_types=(rows_t + srcb_t + dstb_t + valb_t
                       + [pltpu.VMEM((ZROWS, D), jnp.float32),
                          pltpu.VMEM_SHARED((N_NODES, D), jnp.float32)]
                       + sems_t),
    )
    def k(x_hbm, src_hbm, dst_hbm, vals_hbm, out_hbm, *refs):
        rows = refs[0:NBUF]
        srcb = refs[NBUF:2 * NBUF]
        dstb = refs[2 * NBUF:3 * NBUF]
        valb = refs[3 * NBUF:4 * NBUF]
        zero_v = refs[4 * NBUF]
        acc_sh = refs[4 * NBUF + 1]
        rsem = refs[4 * NBUF + 2:4 * NBUF + 2 + NBUF]
        ssem = refs[4 * NBUF + 2 + NBUF:4 * NBUF + 2 + 2 * NBUF]
        msem = refs[4 * NBUF + 2 + 2 * NBUF:4 * NBUF + 2 + 3 * NBUF]

        cid = lax.axis_index("c")
        sid = lax.axis_index("s")
        wid = sid * NC + cid
        base0 = wid * EPW

        def meta_fetch(c, q):
            sl = pl.ds(base0 + c * K, K)
            pltpu.async_copy(src_hbm.at[sl], srcb[q], msem[q])
            pltpu.async_copy(dst_hbm.at[sl], dstb[q], msem[q])
            pltpu.async_copy(vals_hbm.at[sl], valb[q], msem[q])

        def meta_wait(c, q):
            sl = pl.ds(base0 + c * K, K)
            pltpu.make_async_copy(src_hbm.at[sl], srcb[q], msem[q]).wait()
            pltpu.make_async_copy(dst_hbm.at[sl], dstb[q], msem[q]).wait()
            pltpu.make_async_copy(vals_hbm.at[sl], valb[q], msem[q]).wait()

        def gather(q):
            pltpu.async_copy(x_hbm.at[srcb[q]], rows[q], rsem[q])

        def gather_wait(q):
            pltpu.make_async_copy(x_hbm.at[srcb[q]], rows[q],
                                  rsem[q]).wait()

        def scatter(q):
            pltpu.async_copy(rows[q], acc_sh.at[dstb[q]], ssem[q], add=True)

        def scatter_wait(q):
            pltpu.make_async_copy(rows[q], acc_sh.at[dstb[q]],
                                  ssem[q]).wait()

        def scale(q):
            buf = rows[q]
            vbuf = valb[q]

            def body(g, _):
                vv = vbuf[pl.ds(g * L, L)]
                for i in range(L):
                    v = vv[i]
                    e = g * L + i
                    for j in range(D // L):
                        sl = pl.ds(j * L, L)
                        buf[e, sl] = buf[e, sl] * v
                return 0

            lax.fori_loop(0, K // L, body, 0)

        meta_fetch(0, 0)
        meta_fetch(1, 1)

        zvec = jnp.zeros((L,), jnp.float32)
        for j in range(D // L):
            for i in range(ZROWS):
                zero_v[i, pl.ds(j * L, L)] = zvec
        row0 = pl.multiple_of(sid * RPT, 8)

        def zcopy(t, _):
            off = pl.multiple_of(row0 + t * ZROWS, 8)
            pltpu.sync_copy(zero_v, acc_sh.at[pl.ds(off, ZROWS)])
            return 0

        lax.fori_loop(0, RPT // ZROWS, zcopy, 0)

        @pl.when(sid == NS - 1)
        def _zero_tail():
            pltpu.sync_copy(zero_v, acc_sh.at[pl.ds(RPT * NS, TAIL)])

        meta_wait(0, 0)
        gather(0)
        meta_wait(1, 1)
        gather(1)
        plsc.subcore_barrier()

        def step(s, _):
            c0 = s * NBUF
            for q in range(NBUF):
                c = c0 + q
                f = (q + 2) % NBUF

                @pl.when(c + 2 < CHUNKS)
                def _prefetch():
                    meta_fetch(c + 2, f)

                gather_wait(q)
                scale(q)

                @pl.when(c >= 1)
                def _drain_prev():
                    scatter_wait((q + NBUF - 1) % NBUF)

                scatter(q)

                @pl.when(c + 2 < CHUNKS)
                def _launch():
                    meta_wait(c + 2, f)
                    gather(f)

            return 0

        lax.fori_loop(0, STEPS, step, 0)

        last = CHUNKS - 1
        gather_wait(last % NBUF)
        scale(last % NBUF)
        scatter_wait((last - 1) % NBUF)
        scatter(last % NBUF)
        scatter_wait(last % NBUF)
        plsc.subcore_barrier()

        pltpu.sync_copy(acc_sh.at[pl.ds(row0, RPT)],
                        out_hbm.at[cid, pl.ds(row0, RPT)])

        @pl.when(sid == NS - 1)
        def _write_tail():
            pltpu.sync_copy(acc_sh.at[pl.ds(RPT * NS, TAIL)],
                            out_hbm.at[cid, pl.ds(RPT * NS, TAIL)])

    return k(x, src, dst, vals)


def _tc_combine(partials, W, b):
    R = 1000

    def body(p0_ref, p1_ref, w_ref, b_ref, o_ref):
        s = p0_ref[...] + p1_ref[...]
        y = jnp.dot(s, w_ref[...], preferred_element_type=jnp.float32)
        o_ref[...] = jnp.maximum(y + b_ref[...], 0.0)

    return pl.pallas_call(
        body,
        grid=(N_NODES // R,),
        in_specs=[
            pl.BlockSpec((R, D), lambda i: (i, 0)),
            pl.BlockSpec((R, D), lambda i: (i, 0)),
            pl.BlockSpec((D, D), lambda i: (0, 0)),
            pl.BlockSpec((1, D), lambda i: (0, 0)),
        ],
        out_specs=pl.BlockSpec((R, D), lambda i: (i, 0)),
        out_shape=jax.ShapeDtypeStruct((N_NODES, D), jnp.float32),
    )(partials[0], partials[1], W, b.reshape(1, D))


def kernel(x, edge_index, edge_vals, W, b):
    src = edge_index[0].astype(jnp.int32)
    dst = edge_index[1].astype(jnp.int32)
    partials = _sc_scatter(x, src, dst, edge_vals.astype(jnp.float32))
    return _tc_combine(partials, W, b)
